# baseline (speedup 1.0000x reference)
import functools

import jax
import jax.numpy as jnp
from jax.experimental import pallas as pl
from jax.experimental.pallas import tpu as pltpu

_ALPHA = 0.95
_LAMBDA = 1.0
_BLOCK_ROWS = 256
_INT_MIN = -(2 ** 31)


def _cvar_body(nq, x_ref, out_ref, keys_ref):
    i = pl.program_id(0)
    x = x_ref[...]
    rows, cols = x.shape

    bits = jax.lax.bitcast_convert_type(x, jnp.int32)
    keys_ref[...] = bits ^ jnp.bitwise_and(
        jax.lax.shift_right_arithmetic(bits, 31), jnp.int32(0x7FFFFFFF))

    row_sum = jnp.sum(x, axis=1)

    ones = jnp.ones((cols, 1), dtype=jnp.bfloat16)
    nq_f = jnp.float32(nq)

    def step(it, prefix):
        bit = jnp.int32(31) - it
        trial = prefix + jnp.left_shift(jnp.int32(1), bit)
        mask_b = jnp.where(keys_ref[...] < trial, 1.0, 0.0).astype(jnp.bfloat16)
        cnt = jax.lax.dot_general(
            mask_b, ones, (((1,), (0,)), ((), ())),
            preferred_element_type=jnp.float32)
        return jnp.where(cnt < nq_f, trial, prefix)

    prefix0 = jnp.full((rows, 1), _INT_MIN, dtype=jnp.int32)
    t_key = jax.lax.fori_loop(0, 32, step, prefix0)

    mask = keys_ref[...] < t_key
    cnt_less = jnp.sum(mask.astype(jnp.float32), axis=1)
    sum_less = jnp.sum(jnp.where(mask, x, 0.0), axis=1)

    t_bits = t_key ^ jnp.bitwise_and(
        jax.lax.shift_right_arithmetic(t_key, 31), jnp.int32(0x7FFFFFFF))
    t_val = jax.lax.bitcast_convert_type(t_bits, jnp.float32)[:, 0]

    tail_sum = sum_less + t_val * (jnp.float32(nq) - cnt_less)
    loss = -row_sum * jnp.float32(1.0 / cols) + \
        _LAMBDA * tail_sum * jnp.float32(1.0 / nq)
    partial = jnp.sum(loss).reshape(1, 1)

    @pl.when(i == 0)
    def _():
        out_ref[...] = jnp.zeros((1, 1), jnp.float32)

    out_ref[...] += partial


def kernel(pred_rets):
    batch, cols = pred_rets.shape
    nq = int(cols * (1 - _ALPHA))
    if nq == 0:
        nq = 1
    block_rows = min(_BLOCK_ROWS, batch)
    grid = batch // block_rows

    out = pl.pallas_call(
        functools.partial(_cvar_body, nq),
        grid=(grid,),
        in_specs=[pl.BlockSpec((block_rows, cols), lambda i: (i, 0))],
        out_specs=pl.BlockSpec((1, 1), lambda i: (0, 0)),
        out_shape=jax.ShapeDtypeStruct((1, 1), jnp.float32),
        scratch_shapes=[pltpu.VMEM((block_rows, cols), jnp.int32)],
    )(pred_rets)
    return jnp.reshape(out, ()) * jnp.float32(1.0 / batch)


# speedup vs baseline: 1.7438x; 1.7438x over previous
import functools

import jax
import jax.numpy as jnp
from jax.experimental import pallas as pl
from jax.experimental.pallas import tpu as pltpu

_ALPHA = 0.95
_LAMBDA = 1.0
_BLOCK_ROWS = 512
_INT_MIN = -(2 ** 31)


def _cvar_body(nq, x_ref, out_ref, keys_ref):
    i = pl.program_id(0)
    x = x_ref[...]
    rows, cols = x.shape

    bits = jax.lax.bitcast_convert_type(x, jnp.int32)
    keys_ref[...] = bits ^ jnp.bitwise_and(
        jax.lax.shift_right_arithmetic(bits, 31), jnp.int32(0x7FFFFFFF))

    row_sum = jnp.sum(x, axis=1)

    def step(it, prefix):
        bit = jnp.int32(31) - it
        trial = prefix + jnp.left_shift(jnp.int32(1), bit)
        cnt = jnp.sum((keys_ref[...] < trial).astype(jnp.int32), axis=1,
                      keepdims=True)
        return jnp.where(cnt < nq, trial, prefix)

    prefix0 = jnp.full((rows, 1), _INT_MIN, dtype=jnp.int32)
    t_key = jax.lax.fori_loop(0, 32, step, prefix0)

    mask = keys_ref[...] < t_key
    cnt_less = jnp.sum(mask.astype(jnp.float32), axis=1)
    sum_less = jnp.sum(jnp.where(mask, x, 0.0), axis=1)

    t_bits = t_key ^ jnp.bitwise_and(
        jax.lax.shift_right_arithmetic(t_key, 31), jnp.int32(0x7FFFFFFF))
    t_val = jax.lax.bitcast_convert_type(t_bits, jnp.float32)[:, 0]

    tail_sum = sum_less + t_val * (jnp.float32(nq) - cnt_less)
    loss = -row_sum * jnp.float32(1.0 / cols) + \
        _LAMBDA * tail_sum * jnp.float32(1.0 / nq)
    partial = jnp.sum(loss).reshape(1, 1)

    @pl.when(i == 0)
    def _():
        out_ref[...] = jnp.zeros((1, 1), jnp.float32)

    out_ref[...] += partial


def kernel(pred_rets):
    batch, cols = pred_rets.shape
    nq = int(cols * (1 - _ALPHA))
    if nq == 0:
        nq = 1
    block_rows = min(_BLOCK_ROWS, batch)
    grid = batch // block_rows

    out = pl.pallas_call(
        functools.partial(_cvar_body, nq),
        grid=(grid,),
        in_specs=[pl.BlockSpec((block_rows, cols), lambda i: (i, 0))],
        out_specs=pl.BlockSpec((1, 1), lambda i: (0, 0)),
        out_shape=jax.ShapeDtypeStruct((1, 1), jnp.float32),
        scratch_shapes=[pltpu.VMEM((block_rows, cols), jnp.int32)],
    )(pred_rets)
    return jnp.reshape(out, ()) * jnp.float32(1.0 / batch)


# speedup vs baseline: 1.8436x; 1.0572x over previous
import functools

import jax
import jax.numpy as jnp
from jax.experimental import pallas as pl
from jax.experimental.pallas import tpu as pltpu

_ALPHA = 0.95
_LAMBDA = 1.0
_BLOCK_ROWS = 1024
_INT_MIN = -(2 ** 31)


def _cvar_body(nq, x_ref, out_ref, keys_ref):
    i = pl.program_id(0)
    x = x_ref[...]
    rows, cols = x.shape

    bits = jax.lax.bitcast_convert_type(x, jnp.int32)
    keys_ref[...] = bits ^ jnp.bitwise_and(
        jax.lax.shift_right_arithmetic(bits, 31), jnp.int32(0x7FFFFFFF))

    row_sum = jnp.sum(x, axis=1)

    def step(it, prefix):
        bit = jnp.int32(31) - it
        trial = prefix + jnp.left_shift(jnp.int32(1), bit)
        cnt = jnp.sum((keys_ref[...] < trial).astype(jnp.int32), axis=1,
                      keepdims=True)
        return jnp.where(cnt < nq, trial, prefix)

    prefix0 = jnp.full((rows, 1), _INT_MIN, dtype=jnp.int32)
    t_key = jax.lax.fori_loop(0, 32, step, prefix0)

    mask = keys_ref[...] < t_key
    cnt_less = jnp.sum(mask.astype(jnp.float32), axis=1)
    sum_less = jnp.sum(jnp.where(mask, x, 0.0), axis=1)

    t_bits = t_key ^ jnp.bitwise_and(
        jax.lax.shift_right_arithmetic(t_key, 31), jnp.int32(0x7FFFFFFF))
    t_val = jax.lax.bitcast_convert_type(t_bits, jnp.float32)[:, 0]

    tail_sum = sum_less + t_val * (jnp.float32(nq) - cnt_less)
    loss = -row_sum * jnp.float32(1.0 / cols) + \
        _LAMBDA * tail_sum * jnp.float32(1.0 / nq)
    partial = jnp.sum(loss).reshape(1, 1)

    @pl.when(i == 0)
    def _():
        out_ref[...] = jnp.zeros((1, 1), jnp.float32)

    out_ref[...] += partial


def kernel(pred_rets):
    batch, cols = pred_rets.shape
    nq = int(cols * (1 - _ALPHA))
    if nq == 0:
        nq = 1
    block_rows = min(_BLOCK_ROWS, batch)
    grid = batch // block_rows

    out = pl.pallas_call(
        functools.partial(_cvar_body, nq),
        grid=(grid,),
        in_specs=[pl.BlockSpec((block_rows, cols), lambda i: (i, 0))],
        out_specs=pl.BlockSpec((1, 1), lambda i: (0, 0)),
        out_shape=jax.ShapeDtypeStruct((1, 1), jnp.float32),
        scratch_shapes=[pltpu.VMEM((block_rows, cols), jnp.int32)],
    )(pred_rets)
    return jnp.reshape(out, ()) * jnp.float32(1.0 / batch)


# speedup vs baseline: 2.2134x; 1.2006x over previous
import functools

import jax
import jax.numpy as jnp
from jax.experimental import pallas as pl
from jax.experimental.pallas import tpu as pltpu

_ALPHA = 0.95
_LAMBDA = 1.0
_BLOCK_ROWS = 1024
_INT_MIN = -(2 ** 31)


def _cvar_body(nq, x_ref, out_ref, keys_ref, hi_ref, lo_ref):
    i = pl.program_id(0)
    x = x_ref[...]
    rows, cols = x.shape

    bits = jax.lax.bitcast_convert_type(x, jnp.int32)
    keys = bits ^ jnp.bitwise_and(
        jax.lax.shift_right_arithmetic(bits, 31), jnp.int32(0x7FFFFFFF))
    keys_ref[...] = keys
    hi_ref[...] = jax.lax.shift_right_arithmetic(keys, 16).astype(jnp.int16)
    lo_ref[...] = (keys ^ jnp.int32(0x8000)).astype(jnp.int16)

    row_sum = jnp.sum(x, axis=1)
    nq16 = jnp.full((1, 1), nq, dtype=jnp.int16)

    def count16(ref, trial):
        m16 = (ref[...] < trial).astype(jnp.int16)
        s = jnp.sum(pltpu.bitcast(m16, jnp.int32), axis=1, keepdims=True)
        return pltpu.bitcast(s, jnp.int16)

    def step16(ref, k_need, it, prefix):
        delta = jnp.left_shift(jnp.int32(1), jnp.int32(15) - it)
        trial = prefix + jnp.broadcast_to(delta, (1, 1)).astype(jnp.int16)
        return jnp.where(count16(ref, trial) < k_need, trial, prefix)

    pa0 = jnp.full((rows, 1), -32768, dtype=jnp.int16)
    h = jax.lax.fori_loop(0, 16, functools.partial(step16, hi_ref, nq16), pa0)

    k2 = nq16 - count16(hi_ref, h)

    lo_ref[...] = jnp.where(hi_ref[...] == h, lo_ref[...],
                            jnp.full((rows, cols), 32767, dtype=jnp.int16))
    l = jax.lax.fori_loop(0, 16, functools.partial(step16, lo_ref, k2), pa0)

    t_key = jnp.left_shift(h.astype(jnp.int32), 16) | jnp.bitwise_and(
        l.astype(jnp.int32) ^ jnp.int32(0x8000), jnp.int32(0xFFFF))

    mask = keys_ref[...] < t_key
    cnt_less = jnp.sum(mask.astype(jnp.float32), axis=1)
    sum_less = jnp.sum(jnp.where(mask, x, 0.0), axis=1)

    t_bits = t_key ^ jnp.bitwise_and(
        jax.lax.shift_right_arithmetic(t_key, 31), jnp.int32(0x7FFFFFFF))
    t_val = jax.lax.bitcast_convert_type(t_bits, jnp.float32)[:, 0]

    tail_sum = sum_less + t_val * (jnp.float32(nq) - cnt_less)
    loss = -row_sum * jnp.float32(1.0 / cols) + \
        _LAMBDA * tail_sum * jnp.float32(1.0 / nq)
    partial = jnp.sum(loss).reshape(1, 1)

    @pl.when(i == 0)
    def _():
        out_ref[...] = jnp.zeros((1, 1), jnp.float32)

    out_ref[...] += partial


def kernel(pred_rets):
    batch, cols = pred_rets.shape
    nq = int(cols * (1 - _ALPHA))
    if nq == 0:
        nq = 1
    block_rows = min(_BLOCK_ROWS, batch)
    grid = batch // block_rows

    out = pl.pallas_call(
        functools.partial(_cvar_body, nq),
        grid=(grid,),
        in_specs=[pl.BlockSpec((block_rows, cols), lambda i: (i, 0))],
        out_specs=pl.BlockSpec((1, 1), lambda i: (0, 0)),
        out_shape=jax.ShapeDtypeStruct((1, 1), jnp.float32),
        scratch_shapes=[pltpu.VMEM((block_rows, cols), jnp.int32),
                        pltpu.VMEM((block_rows, cols), jnp.int16),
                        pltpu.VMEM((block_rows, cols), jnp.int16)],
    )(pred_rets)
    return jnp.reshape(out, ()) * jnp.float32(1.0 / batch)


# speedup vs baseline: 2.2421x; 1.0130x over previous
import functools

import jax
import jax.numpy as jnp
from jax.experimental import pallas as pl
from jax.experimental.pallas import tpu as pltpu

_ALPHA = 0.95
_LAMBDA = 1.0
_BLOCK_ROWS = 1024
_INT_MIN = -(2 ** 31)


def _keys_of(x):
    bits = jax.lax.bitcast_convert_type(x, jnp.int32)
    return bits ^ jnp.bitwise_and(
        jax.lax.shift_right_arithmetic(bits, 31), jnp.int32(0x7FFFFFFF))


def _cvar_body(nq, x_ref, out_ref, hi_ref, lo_ref):
    i = pl.program_id(0)
    x = x_ref[...]
    rows, cols = x.shape

    keys = _keys_of(x)
    hi_ref[...] = jax.lax.shift_right_arithmetic(keys, 16).astype(jnp.int16)
    lo_ref[...] = (keys ^ jnp.int32(0x8000)).astype(jnp.int16)

    row_sum = jnp.sum(x, axis=1)
    nq16 = jnp.full((1, 1), nq, dtype=jnp.int16)

    def count16(ref, trial):
        m16 = (ref[...] < trial).astype(jnp.int16)
        s = jnp.sum(pltpu.bitcast(m16, jnp.int32), axis=1, keepdims=True)
        return pltpu.bitcast(s, jnp.int16)

    def step16(ref, k_need, it, prefix):
        delta = jnp.left_shift(jnp.int32(1), jnp.int32(15) - it)
        trial = prefix + jnp.broadcast_to(delta, (1, 1)).astype(jnp.int16)
        return jnp.where(count16(ref, trial) < k_need, trial, prefix)

    pa0 = jnp.full((rows, 1), -32768, dtype=jnp.int16)
    h = jax.lax.fori_loop(0, 16, functools.partial(step16, hi_ref, nq16), pa0)

    k2 = nq16 - count16(hi_ref, h)

    lo_ref[...] = jnp.where(hi_ref[...] == h, lo_ref[...],
                            jnp.full((rows, cols), 32767, dtype=jnp.int16))
    l = jax.lax.fori_loop(0, 16, functools.partial(step16, lo_ref, k2), pa0)

    t_key = jnp.left_shift(h.astype(jnp.int32), 16) | jnp.bitwise_and(
        l.astype(jnp.int32) ^ jnp.int32(0x8000), jnp.int32(0xFFFF))

    cnt_less = ((nq16 - k2) + count16(lo_ref, l)).astype(jnp.float32)[:, 0]
    mask = _keys_of(x_ref[...]) < t_key
    sum_less = jnp.sum(jnp.where(mask, x_ref[...], 0.0), axis=1)

    t_bits = t_key ^ jnp.bitwise_and(
        jax.lax.shift_right_arithmetic(t_key, 31), jnp.int32(0x7FFFFFFF))
    t_val = jax.lax.bitcast_convert_type(t_bits, jnp.float32)[:, 0]

    tail_sum = sum_less + t_val * (jnp.float32(nq) - cnt_less)
    loss = -row_sum * jnp.float32(1.0 / cols) + \
        _LAMBDA * tail_sum * jnp.float32(1.0 / nq)
    partial = jnp.sum(loss).reshape(1, 1)

    @pl.when(i == 0)
    def _():
        out_ref[...] = jnp.zeros((1, 1), jnp.float32)

    out_ref[...] += partial


def kernel(pred_rets):
    batch, cols = pred_rets.shape
    nq = int(cols * (1 - _ALPHA))
    if nq == 0:
        nq = 1
    block_rows = min(_BLOCK_ROWS, batch)
    grid = batch // block_rows

    out = pl.pallas_call(
        functools.partial(_cvar_body, nq),
        grid=(grid,),
        in_specs=[pl.BlockSpec((block_rows, cols), lambda i: (i, 0))],
        out_specs=pl.BlockSpec((1, 1), lambda i: (0, 0)),
        out_shape=jax.ShapeDtypeStruct((1, 1), jnp.float32),
        scratch_shapes=[pltpu.VMEM((block_rows, cols), jnp.int16),
                        pltpu.VMEM((block_rows, cols), jnp.int16)],
    )(pred_rets)
    return jnp.reshape(out, ()) * jnp.float32(1.0 / batch)
